# Initial kernel scaffold; baseline (speedup 1.0000x reference)
import jax
import jax.numpy as jnp
from jax.experimental import pallas as pl


def kernel(x, edge_index):
    raise NotImplementedError("write your pallas kernel here")


# speedup vs baseline: 1.2415x; 1.2415x over previous
import functools

import jax
import jax.numpy as jnp
from jax import lax
from jax.experimental import pallas as pl
from jax.experimental.pallas import tpu as pltpu
from jax.experimental.pallas import tpu_sc as plsc

N_NODES = 10000
N_EDGES = 320000
D = 128
L = 16
NC, NS = 2, 16
NW = NC * NS
N_PAD = 10240
ROWS = N_PAD // NW
CHUNK = 4000
VPC = CHUNK // L
NCHUNK = N_EDGES // CHUNK
QCAP = 128
QPAD = QCAP + L


TPAD = ROWS + L


def _body(x_hbm, src_hbm, dst_hbm, out_hbm,
          acc, gbuf, srcc, dstc, qs, qd, touched, sem):
  wid = lax.axis_index("s") * NC + lax.axis_index("c")
  lo = wid * ROWS
  hi = lo + ROWS

  zeros = jnp.zeros((L,), jnp.int32)
  neg_inf = jnp.full((L,), -jnp.inf, jnp.float32)

  def init_acc(r, _):
    for j in range(D // L):
      acc[r, pl.ds(j * L, L)] = neg_inf
    return 0
  lax.fori_loop(0, ROWS, init_acc, 0)

  def init_t(i, _):
    touched[pl.ds(i * L, L)] = zeros
    return 0
  lax.fori_loop(0, TPAD // L, init_t, 0)
  for i in range(QCAP // L):
    qs[pl.ds(i * L, L)] = zeros
  for i in range(QPAD // L):
    qd[pl.ds(i * L, L)] = zeros

  one_vec = jnp.ones((L,), jnp.int32)
  lane0 = lax.iota(jnp.int32, L) == 0

  def flush(k):
    pltpu.async_copy(x_hbm.at[qs], gbuf, sem).wait()

    def edge_body(e, _):
      d = qd[pl.ds(e, L)][0]
      r = d - lo
      for j in range(D // L):
        sl = pl.ds(j * L, L)
        acc[r, sl] = jnp.maximum(acc[r, sl], gbuf[e, sl])
      plsc.store_scatter(touched, [jnp.full((L,), r, jnp.int32)], one_vec,
                         mask=lane0)
      return 0
    lax.fori_loop(0, k, edge_body, 0)

  def vec_body(v, pos):
    def do_flush(p):
      flush(p)
      return 0
    pos = lax.cond(pos > QCAP - L, do_flush, lambda p: p, pos)

    s = srcc[pl.ds(v * L, L)]
    dv = dstc[pl.ds(v * L, L)]
    m = (dv >= lo) & (dv < hi)
    cs = plsc.cumsum(m.astype(jnp.int32))
    idx = pos + cs - 1
    plsc.store_scatter(qs, [idx], s, mask=m)
    plsc.store_scatter(qd, [idx], dv, mask=m)
    return pos + cs[L - 1]

  def chunk_body(c, pos):
    off = pl.multiple_of(c * CHUNK, 8)
    pltpu.sync_copy(src_hbm.at[pl.ds(off, CHUNK)], srcc)
    pltpu.sync_copy(dst_hbm.at[pl.ds(off, CHUNK)], dstc)
    return lax.fori_loop(0, VPC, vec_body, pos)

  pos = lax.fori_loop(0, NCHUNK, chunk_body, 0)

  @pl.when(pos > 0)
  def _():
    flush(pos)

  for base, nrows in ((0, QCAP), (QCAP, QCAP), (2 * QCAP, ROWS - 2 * QCAP)):
    pltpu.sync_copy(x_hbm.at[pl.ds(lo + base, nrows)], gbuf.at[pl.ds(0, nrows)])

    def fix_body(r, _, base=base):
      @pl.when(touched[pl.ds(base + r, L)][0] == 0)
      def _():
        for j in range(D // L):
          sl = pl.ds(j * L, L)
          acc[base + r, sl] = gbuf[r, sl]
      return 0
    lax.fori_loop(0, nrows, fix_body, 0)
    pltpu.sync_copy(acc.at[pl.ds(base, nrows)],
                    out_hbm.at[pl.ds(lo + base, nrows)])


@jax.jit
def _gcn_sc(x_pad, src, dst):
  mesh = plsc.VectorSubcoreMesh(core_axis_name="c", subcore_axis_name="s",
                                num_cores=NC, num_subcores=NS)
  return pl.kernel(
      _body,
      out_type=jax.ShapeDtypeStruct((N_PAD, D), jnp.float32),
      mesh=mesh,
      compiler_params=pltpu.CompilerParams(needs_layout_passes=False),
      scratch_types=[
          pltpu.VMEM((ROWS, D), jnp.float32),
          pltpu.VMEM((QCAP, D), jnp.float32),
          pltpu.VMEM((CHUNK,), jnp.int32),
          pltpu.VMEM((CHUNK,), jnp.int32),
          pltpu.VMEM((QCAP,), jnp.int32),
          pltpu.VMEM((QPAD,), jnp.int32),
          pltpu.VMEM((TPAD,), jnp.int32),
          pltpu.SemaphoreType.DMA,
      ],
  )(x_pad, src, dst)


def kernel(x, edge_index):
  x_pad = jnp.zeros((N_PAD, D), jnp.float32).at[:N_NODES].set(x)
  v = _gcn_sc(x_pad, edge_index[0], edge_index[1])
  return jnp.concatenate([x, v[:N_NODES]], axis=1)


# speedup vs baseline: 1.3015x; 1.0483x over previous
import functools

import jax
import jax.numpy as jnp
from jax import lax
from jax.experimental import pallas as pl
from jax.experimental.pallas import tpu as pltpu
from jax.experimental.pallas import tpu_sc as plsc

N_NODES = 10000
N_EDGES = 320000
D = 128
L = 16
NC, NS = 2, 16
N_PAD = 10240
NGRP = 16
ROWS = N_PAD // NGRP
HALF = ROWS // 2
EHALF = N_EDGES // 2
CHUNK = 4000
VPC = CHUNK // L
NCHUNK = EHALF // CHUNK
QCAP = 128
QPAD = QCAP + L
TPAD = ROWS + L


def _body(x_hbm, src_hbm, dst_hbm, out_hbm,
          acc, gbuf, srcc, dstc, qs, qd, touched, tpart, xch, tch, sem):
  c = lax.axis_index("c")
  s = lax.axis_index("s")
  member = s // 8
  pair = s % 8
  grp = c * 8 + pair
  glo = grp * ROWS
  ghi = glo + ROWS

  zeros = jnp.zeros((L,), jnp.int32)
  neg_inf = jnp.full((L,), -jnp.inf, jnp.float32)
  one_vec = jnp.ones((L,), jnp.int32)
  lane0 = lax.iota(jnp.int32, L) == 0

  def init_acc(r, _):
    for j in range(D // L):
      acc[r, pl.ds(j * L, L)] = neg_inf
    return 0
  lax.fori_loop(0, ROWS, init_acc, 0)

  def init_t(i, _):
    touched[pl.ds(i * L, L)] = zeros
    return 0
  lax.fori_loop(0, TPAD // L, init_t, 0)
  for i in range(QCAP // L):
    qs[pl.ds(i * L, L)] = zeros

  def flush(k):
    pltpu.async_copy(x_hbm.at[qs], gbuf, sem).wait()

    def edge_body(e, _):
      d = qd[pl.ds(e, L)][0]
      r = d - glo
      for j in range(D // L):
        sl = pl.ds(j * L, L)
        acc[r, sl] = jnp.maximum(acc[r, sl], gbuf[e, sl])
      plsc.store_scatter(touched, [jnp.full((L,), r, jnp.int32)], one_vec,
                         mask=lane0)
      return 0
    lax.fori_loop(0, k, edge_body, 0)

  def vec_body(v, pos):
    dv = dstc[pl.ds(v * L, L)]
    m = (dv >= glo) & (dv < ghi)

    def hit(p):
      def do_flush(q):
        flush(q)
        return 0
      p = lax.cond(p > QCAP - L, do_flush, lambda q: q, p)
      sv = srcc[pl.ds(v * L, L)]
      cs = plsc.cumsum(m.astype(jnp.int32))
      idx = p + cs - 1
      plsc.store_scatter(qs, [idx], sv, mask=m)
      plsc.store_scatter(qd, [idx], dv, mask=m)
      return p + cs[L - 1]

    return lax.cond(jnp.any(m), hit, lambda p: p, pos)

  def chunk_body(ch, pos):
    off = pl.multiple_of(member * EHALF + ch * CHUNK, 8)
    pltpu.sync_copy(src_hbm.at[pl.ds(off, CHUNK)], srcc)
    pltpu.sync_copy(dst_hbm.at[pl.ds(off, CHUNK)], dstc)
    return lax.fori_loop(0, VPC, vec_body, pos)

  pos = lax.fori_loop(0, NCHUNK, chunk_body, 0)

  @pl.when(pos > 0)
  def _():
    flush(pos)

  other = 1 - member
  slot = pair * 2 + member
  pslot = pair * 2 + other
  base = member * HALF

  pltpu.sync_copy(touched.at[pl.ds(other * HALF, HALF)],
                  tch.at[pl.ds(pl.multiple_of(slot * HALF, 8), HALF)])
  for mb, mrows in ((0, QCAP), (QCAP, QCAP), (2 * QCAP, HALF - 2 * QCAP)):
    pltpu.sync_copy(acc.at[pl.ds(other * HALF + mb, mrows)],
                    xch.at[slot, pl.ds(0, mrows)])
    plsc.subcore_barrier()
    pltpu.sync_copy(xch.at[pslot, pl.ds(0, mrows)], gbuf.at[pl.ds(0, mrows)])

    def merge_body(r, _, mb=mb):
      for j in range(D // L):
        sl = pl.ds(j * L, L)
        acc[base + mb + r, sl] = jnp.maximum(acc[base + mb + r, sl],
                                             gbuf[r, sl])
      return 0
    lax.fori_loop(0, mrows, merge_body, 0)
    plsc.subcore_barrier()

  pltpu.sync_copy(tch.at[pl.ds(pl.multiple_of(pslot * HALF, 8), HALF)],
                  tpart.at[pl.ds(0, HALF)])

  def morrow(r, _):
    @pl.when(tpart[pl.ds(r, L)][0] > 0)
    def _():
      plsc.store_scatter(touched, [jnp.full((L,), base + r, jnp.int32)],
                         one_vec, mask=lane0)
    return 0
  lax.fori_loop(0, HALF, morrow, 0)

  for wb, wrows in ((0, QCAP), (QCAP, QCAP), (2 * QCAP, HALF - 2 * QCAP)):
    pltpu.sync_copy(x_hbm.at[pl.ds(glo + base + wb, wrows)],
                    gbuf.at[pl.ds(0, wrows)])

    def fix_body(r, _, wb=wb):
      @pl.when(touched[pl.ds(base + wb + r, L)][0] == 0)
      def _():
        for j in range(D // L):
          sl = pl.ds(j * L, L)
          acc[base + wb + r, sl] = gbuf[r, sl]
      return 0
    lax.fori_loop(0, wrows, fix_body, 0)
    pltpu.sync_copy(acc.at[pl.ds(base + wb, wrows)],
                    out_hbm.at[pl.ds(glo + base + wb, wrows)])


@jax.jit
def _gcn_sc(x_pad, src, dst):
  mesh = plsc.VectorSubcoreMesh(core_axis_name="c", subcore_axis_name="s",
                                num_cores=NC, num_subcores=NS)
  return pl.kernel(
      _body,
      out_type=jax.ShapeDtypeStruct((N_PAD, D), jnp.float32),
      mesh=mesh,
      compiler_params=pltpu.CompilerParams(needs_layout_passes=False),
      scratch_types=[
          pltpu.VMEM((ROWS, D), jnp.float32),
          pltpu.VMEM((QCAP, D), jnp.float32),
          pltpu.VMEM((CHUNK,), jnp.int32),
          pltpu.VMEM((CHUNK,), jnp.int32),
          pltpu.VMEM((QCAP,), jnp.int32),
          pltpu.VMEM((QPAD,), jnp.int32),
          pltpu.VMEM((TPAD,), jnp.int32),
          pltpu.VMEM((HALF + L,), jnp.int32),
          pltpu.VMEM_SHARED((NS, QCAP, D), jnp.float32),
          pltpu.VMEM_SHARED((NS * HALF,), jnp.int32),
          pltpu.SemaphoreType.DMA,
      ],
  )(x_pad, src, dst)


def kernel(x, edge_index):
  x_pad = jnp.zeros((N_PAD, D), jnp.float32).at[:N_NODES].set(x)
  v = _gcn_sc(x_pad, edge_index[0], edge_index[1])
  return jnp.concatenate([x, v[:N_NODES]], axis=1)
